# baseline (speedup 1.0000x reference)
import functools

import jax
import jax.numpy as jnp
from jax import lax
from jax.experimental import pallas as pl
from jax.experimental.pallas import tpu as pltpu
from jax.experimental.pallas import tpu_sc as plsc

NC = 2
NS = 16
L = 16
NW = NC * NS

ROWS = 16384
COLS = 200
JT = COLS // 8
IHI = ROWS // 128
SLABS = JT * IHI
NB = 5
NT = 5

_mesh = plsc.VectorSubcoreMesh(core_axis_name="c", subcore_axis_name="s")


@functools.partial(
    pl.kernel,
    mesh=_mesh,
    out_type=jax.ShapeDtypeStruct((COLS, IHI, 128), jnp.float32),
    scratch_types=[
        pltpu.VMEM((2, L), jnp.float32),
        pltpu.VMEM((NB, 4, 8, 128), jnp.int32),
        pltpu.VMEM((NB, 8, 4, 128), jnp.float32),
        [pltpu.SemaphoreType.DMA] * (2 * NB),
    ],
)
def _emb_lookup(idx_hbm, w_hbm, out_hbm, w_v, idx_v, out_v, sems):
    wid = lax.axis_index("s") * NC + lax.axis_index("c")
    u0 = wid * NB * NT
    s_in, s_out = sems[:NB], sems[NB:]

    pltpu.sync_copy(w_hbm, w_v)
    w0 = w_v[0, :]
    d = w_v[1, :] - w0

    def in_copy(u, m):
        return pltpu.make_async_copy(
            idx_hbm.at[pl.ds(u * 4, 4), :, :], idx_v.at[m], s_in[m])

    def out_copy(u, m):
        jt = u // (IHI // 4)
        g = u % (IHI // 4)
        return pltpu.make_async_copy(
            out_v.at[m],
            out_hbm.at[pl.ds(jt * 8, 8), pl.ds(g * 4, 4), :], s_out[m])

    def round_body(t, _):
        for m in range(NB):
            in_copy(u0 + t * NB + m, m).start()
        for m in range(NB):
            u = u0 + t * NB + m
            in_copy(u, m).wait()

            @pl.when(t > 0)
            def _():
                out_copy(u - NB, m).wait()

            @plsc.parallel_loop(0, 4, step=1)
            def _unit(r):
                for jj in range(8):
                    for off in range(0, 128, L):
                        x = idx_v[m, r, jj, pl.ds(off, L)]
                        out_v[m, jj, r, pl.ds(off, L)] = (
                            w0 + d * x.astype(jnp.float32))

            out_copy(u, m).start()
        return 0

    lax.fori_loop(0, NT, round_body, 0)
    for m in range(NB):
        out_copy(u0 + (NT - 1) * NB + m, m).wait()


def kernel(idx, weight):
    idx3 = (idx.T.reshape(JT, 8, IHI, 128)
            .transpose(0, 2, 1, 3).reshape(SLABS, 8, 128))
    wb = jnp.broadcast_to(weight.astype(jnp.float32), (2, L))
    out3 = _emb_lookup(idx3, wb)
    return out3.transpose(1, 2, 0).reshape(ROWS, COLS, 1)


# speedup vs baseline: 1.3761x; 1.3761x over previous
import functools

import jax
import jax.numpy as jnp
from jax import lax
from jax.experimental import pallas as pl
from jax.experimental.pallas import tpu as pltpu
from jax.experimental.pallas import tpu_sc as plsc

NC = 2
NS = 16
L = 16
NW = NC * NS

ROWS = 16384
COLS = 200
JT = COLS // 8
IHI = ROWS // 128
SLABS = JT * IHI
NB = 5
NT = 5

_mesh = plsc.VectorSubcoreMesh(core_axis_name="c", subcore_axis_name="s")


@functools.partial(
    pl.kernel,
    mesh=_mesh,
    out_type=jax.ShapeDtypeStruct((COLS, IHI, 128), jnp.float32),
    scratch_types=[
        pltpu.VMEM((2, L), jnp.float32),
        pltpu.VMEM((NB, 8, 4, 128), jnp.int32),
        pltpu.VMEM((NB, 4, 8, 128), jnp.float32),
        [pltpu.SemaphoreType.DMA] * (2 * NB),
    ],
)
def _emb_lookup(idx_hbm, w_hbm, out_hbm, w_v, idx_v, out_v, sems):
    wid = lax.axis_index("s") * NC + lax.axis_index("c")
    u0 = wid * NB * NT
    s_in, s_out = sems[:NB], sems[NB:]

    pltpu.sync_copy(w_hbm, w_v)
    w0 = w_v[0, :]
    d = w_v[1, :] - w0

    def _split(u):
        jt = u // 32
        jq = (u // 16) % 2
        h = u % 16
        return jt, jq, h

    def in_copy(u, m):
        jt, jq, h = _split(u)
        return pltpu.make_async_copy(
            idx_hbm.at[pl.ds(jt * 128 + h * 8, 8), pl.ds(jq * 4, 4), :],
            idx_v.at[m], s_in[m])

    def out_copy(u, m):
        jt, jq, h = _split(u)
        return pltpu.make_async_copy(
            out_v.at[m],
            out_hbm.at[pl.ds(jt * 8 + jq * 4, 4), pl.ds(h * 8, 8), :],
            s_out[m])

    def round_body(t, _):
        for m in range(NB):
            in_copy(u0 + t * NB + m, m).start()
        for m in range(NB):
            u = u0 + t * NB + m
            in_copy(u, m).wait()

            @pl.when(t > 0)
            def _():
                out_copy(u - NB, m).wait()

            @plsc.parallel_loop(0, 8, step=1)
            def _unit(r):
                for jj in range(4):
                    for off in range(0, 128, L):
                        x = idx_v[m, r, jj, pl.ds(off, L)]
                        out_v[m, jj, r, pl.ds(off, L)] = (
                            w0 + d * x.astype(jnp.float32))

            out_copy(u, m).start()
        return 0

    lax.fori_loop(0, NT, round_body, 0)
    for m in range(NB):
        out_copy(u0 + (NT - 1) * NB + m, m).wait()


def kernel(idx, weight):
    idx3 = (idx.T.reshape(JT, 8, IHI, 128)
            .transpose(0, 2, 1, 3).reshape(SLABS, 8, 128))
    wb = jnp.broadcast_to(weight.astype(jnp.float32), (2, L))
    out3 = _emb_lookup(idx3, wb)
    return out3.transpose(1, 2, 0).reshape(ROWS, COLS, 1)


# speedup vs baseline: 1.7488x; 1.2709x over previous
import functools

import jax
import jax.numpy as jnp
from jax import lax
from jax.experimental import pallas as pl
from jax.experimental.pallas import tpu as pltpu
from jax.experimental.pallas import tpu_sc as plsc

NC = 2
NS = 16
L = 16
NW = NC * NS

ROWS = 16384
COLS = 200
JT = COLS // 8
IHI = ROWS // 128
SLABS = JT * IHI
NB = 5
NT = 5

_mesh = plsc.VectorSubcoreMesh(core_axis_name="c", subcore_axis_name="s")


@functools.partial(
    pl.kernel,
    mesh=_mesh,
    out_type=jax.ShapeDtypeStruct((SLABS, 8, 128), jnp.float32),
    scratch_types=[
        pltpu.VMEM((2, L), jnp.float32),
        pltpu.VMEM((NB, 4, 8, 128), jnp.int32),
        pltpu.VMEM((NB, 4, 8, 128), jnp.float32),
        [pltpu.SemaphoreType.DMA] * (2 * NB),
    ],
)
def _emb_lookup(idx_hbm, w_hbm, out_hbm, w_v, idx_v, out_v, sems):
    wid = lax.axis_index("s") * NC + lax.axis_index("c")
    u0 = wid * NB * NT
    s_in, s_out = sems[:NB], sems[NB:]

    pltpu.sync_copy(w_hbm, w_v)
    w0 = w_v[0, :]
    d = w_v[1, :] - w0

    def in_copy(u, m):
        return pltpu.make_async_copy(
            idx_hbm.at[pl.ds(u * 4, 4), :, :], idx_v.at[m], s_in[m])

    def out_copy(u, m):
        return pltpu.make_async_copy(
            out_v.at[m], out_hbm.at[pl.ds(u * 4, 4), :, :], s_out[m])

    def round_body(t, _):
        for m in range(NB):
            in_copy(u0 + t * NB + m, m).start()
        for m in range(NB):
            u = u0 + t * NB + m
            in_copy(u, m).wait()

            @pl.when(t > 0)
            def _():
                out_copy(u - NB, m).wait()

            @plsc.parallel_loop(0, 4, step=1)
            def _unit(r):
                for jj in range(8):
                    for off in range(0, 128, L):
                        x = idx_v[m, r, jj, pl.ds(off, L)]
                        out_v[m, r, jj, pl.ds(off, L)] = (
                            w0 + d * x.astype(jnp.float32))

            out_copy(u, m).start()
        return 0

    lax.fori_loop(0, NT, round_body, 0)
    for m in range(NB):
        out_copy(u0 + (NT - 1) * NB + m, m).wait()


def kernel(idx, weight):
    idx3 = (idx.T.reshape(JT, 8, IHI, 128)
            .transpose(0, 2, 1, 3).reshape(SLABS, 8, 128))
    wb = jnp.broadcast_to(weight.astype(jnp.float32), (2, L))
    out3 = _emb_lookup(idx3, wb)
    return (out3.reshape(JT, IHI, 8, 128).transpose(0, 2, 1, 3)
            .reshape(COLS, ROWS).T.reshape(ROWS, COLS, 1))
